# baseline (speedup 1.0000x reference)
import functools

import jax
import jax.numpy as jnp
from jax import lax
from jax.experimental import pallas as pl
from jax.experimental.pallas import tpu as pltpu
from jax.experimental.pallas import tpu_sc as plsc

L = 16
NC, NS = 2, 16
NW = NC * NS
CHUNK = 512
IDXROW = 128
KSTREAM = CHUNK // IDXROW
PACKW = 64
OUTW = 128


def _bcast_lane(v, k):
    return lax.gather(
        v, jnp.full((L, 1), k, jnp.int32),
        lax.GatherDimensionNumbers(
            offset_dims=(), collapsed_slice_dims=(0,), start_index_map=(0,)),
        slice_sizes=(1,),
        mode=lax.GatherScatterMode.PROMISE_IN_BOUNDS)


@functools.lru_cache(maxsize=None)
def _build(n_total, vocab, dim):
    per_w = n_total // NW
    n_chunks = per_w // CHUNK
    pack = OUTW // dim
    out_rows = n_total // pack
    orows_chunk = CHUNK // pack
    max_idx = vocab - 1

    mesh = plsc.VectorSubcoreMesh(core_axis_name="c", subcore_axis_name="s")

    @functools.partial(
        pl.kernel,
        out_type=jax.ShapeDtypeStruct((out_rows, OUTW), jnp.float32),
        mesh=mesh,
        compiler_params=pltpu.CompilerParams(use_tc_tiling_on_sc=False),
        scratch_types=[
            pltpu.VMEM((CHUNK,), jnp.float32),
            pltpu.VMEM((CHUNK,), jnp.float32),
            pltpu.VMEM((KSTREAM, IDXROW), jnp.int32),
            pltpu.VMEM((KSTREAM, IDXROW), jnp.int32),
            pltpu.VMEM((CHUNK, PACKW), jnp.float32),
            pltpu.VMEM((CHUNK, PACKW), jnp.float32),
            pltpu.VMEM((orows_chunk, OUTW), jnp.float32),
            pltpu.SemaphoreType.DMA,
        ],
    )
    def impl(cont_hbm, w_hbm, out_hbm, x_v, w_v, idx_lo_v, idx_hi_v,
             lo_v, hi_v, out_v, sem):
        wid = lax.axis_index("s") * NC + lax.axis_index("c")

        def chunk_body(g, _):
            base = wid * per_w + g * CHUNK
            pltpu.sync_copy(cont_hbm.at[pl.ds(base, CHUNK)], x_v)

            def idx_body(t, _):
                x = x_v[pl.ds(t * L, L)]
                il = x.astype(jnp.int32)
                w = x - il.astype(jnp.float32)
                ih = jnp.minimum(il + 1, max_idx)
                r = t // (IDXROW // L)
                c = (t % (IDXROW // L)) * L
                idx_lo_v[r, pl.ds(c, L)] = il >> 1
                idx_hi_v[r, pl.ds(c, L)] = ih >> 1
                w_v[pl.ds(t * L, L)] = w
                return 0

            lax.fori_loop(0, CHUNK // L, idx_body, 0)

            copies = []
            for j in range(KSTREAM):
                copies.append(pltpu.async_copy(
                    w_hbm.at[idx_lo_v.at[j]],
                    lo_v.at[pl.ds(j * IDXROW, IDXROW)], sem))
                copies.append(pltpu.async_copy(
                    w_hbm.at[idx_hi_v.at[j]],
                    hi_v.at[pl.ds(j * IDXROW, IDXROW)], sem))
            for cp in copies:
                cp.wait()

            def blend_body(t, _):
                x16 = x_v[pl.ds(t * L, L)]
                w16 = w_v[pl.ds(t * L, L)]
                for k in range(L):
                    i = t * L + k
                    orow = t * (L // pack) + k // pack
                    ocol = (k % pack) * dim
                    wv = _bcast_lane(w16, k)
                    xv = _bcast_lane(x16, k)
                    pf = (xv.astype(jnp.int32) & 1).astype(jnp.float32)
                    for d in range(dim // L):
                        a0 = lo_v[i, pl.ds(d * L, L)]
                        a1 = lo_v[i, pl.ds(dim + d * L, L)]
                        b0 = hi_v[i, pl.ds(d * L, L)]
                        b1 = hi_v[i, pl.ds(dim + d * L, L)]
                        lo = a0 + pf * (a1 - a0)
                        hi = b1 + pf * (b0 - b1)
                        out_v[orow, pl.ds(ocol + d * L, L)] = lo + wv * (hi - lo)
                return 0

            lax.fori_loop(0, CHUNK // L, blend_body, 0)

            pltpu.sync_copy(out_v, out_hbm.at[pl.ds(base // pack, orows_chunk)])
            return 0

        lax.fori_loop(0, n_chunks, chunk_body, 0)

    return impl


def kernel(continuous_idx, W):
    batch, fields = continuous_idx.shape
    vocab, dim = W.shape
    n_total = batch * fields
    impl = _build(n_total, vocab, dim)
    out = impl(continuous_idx.reshape(n_total),
               W.reshape(vocab // 2, 2 * dim))
    return out.reshape(batch, fields, dim)


# speedup vs baseline: 1.4575x; 1.4575x over previous
import functools

import jax
import jax.numpy as jnp
from jax import lax
from jax.experimental import pallas as pl
from jax.experimental.pallas import tpu as pltpu
from jax.experimental.pallas import tpu_sc as plsc

L = 16
NC, NS = 2, 16
NW = NC * NS
CHUNK = 512
IDXROW = 128
KSTREAM = CHUNK // IDXROW


def _bcast_lane(v, k):
    return lax.gather(
        v, jnp.full((L, 1), k, jnp.int32),
        lax.GatherDimensionNumbers(
            offset_dims=(), collapsed_slice_dims=(0,), start_index_map=(0,)),
        slice_sizes=(1,),
        mode=lax.GatherScatterMode.PROMISE_IN_BOUNDS)


@functools.lru_cache(maxsize=None)
def _build(n_total, vocab, dim):
    per_w = n_total // NW
    n_chunks = per_w // CHUNK
    max_idx = vocab - 1

    mesh = plsc.VectorSubcoreMesh(core_axis_name="c", subcore_axis_name="s")

    @functools.partial(
        pl.kernel,
        out_type=jax.ShapeDtypeStruct((n_total, dim), jnp.float32),
        mesh=mesh,
        compiler_params=pltpu.CompilerParams(use_tc_tiling_on_sc=False),
        scratch_types=[
            pltpu.VMEM((CHUNK,), jnp.float32),
            pltpu.VMEM((CHUNK,), jnp.float32),
            pltpu.VMEM((KSTREAM, IDXROW), jnp.int32),
            pltpu.VMEM((KSTREAM, IDXROW), jnp.int32),
            pltpu.VMEM((KSTREAM, IDXROW), jnp.int32),
            pltpu.VMEM((KSTREAM, IDXROW), jnp.int32),
            pltpu.VMEM((CHUNK, dim), jnp.float32),
            pltpu.VMEM((CHUNK, dim), jnp.float32),
            pltpu.VMEM((CHUNK, dim), jnp.float32),
            pltpu.VMEM((CHUNK, dim), jnp.float32),
            pltpu.VMEM((CHUNK, dim), jnp.float32),
            pltpu.VMEM((CHUNK, dim), jnp.float32),
            pltpu.SemaphoreType.DMA,
            pltpu.SemaphoreType.DMA,
        ],
    )
    def impl(cont_hbm, w_hbm, out_hbm, c0, c1, il0, ih0, il1, ih1,
             lo0, hi0, lo1, hi1, o0, o1, s0, s1):
        wid = lax.axis_index("s") * NC + lax.axis_index("c")

        def prep(g, cv, ilv, ihv, lov, hiv, sem):
            @pl.when(g < n_chunks)
            def _():
                base = wid * per_w + g * CHUNK
                pltpu.sync_copy(cont_hbm.at[pl.ds(base, CHUNK)], cv)

                def idx_body(t, _):
                    x = cv[pl.ds(t * L, L)]
                    il = x.astype(jnp.int32)
                    w = x - il.astype(jnp.float32)
                    ih = jnp.minimum(il + 1, max_idx)
                    r = t // (IDXROW // L)
                    c = (t % (IDXROW // L)) * L
                    ilv[r, pl.ds(c, L)] = il
                    ihv[r, pl.ds(c, L)] = ih
                    cv[pl.ds(t * L, L)] = w
                    return 0

                lax.fori_loop(0, CHUNK // L, idx_body, 0)
                for j in range(KSTREAM):
                    pltpu.async_copy(
                        w_hbm.at[ilv.at[j]],
                        lov.at[pl.ds(j * IDXROW, IDXROW)], sem)
                    pltpu.async_copy(
                        w_hbm.at[ihv.at[j]],
                        hiv.at[pl.ds(j * IDXROW, IDXROW)], sem)

        def waitg(ilv, ihv, lov, hiv, sem):
            for j in range(KSTREAM):
                pltpu.make_async_copy(
                    w_hbm.at[ilv.at[j]],
                    lov.at[pl.ds(j * IDXROW, IDXROW)], sem).wait()
                pltpu.make_async_copy(
                    w_hbm.at[ihv.at[j]],
                    hiv.at[pl.ds(j * IDXROW, IDXROW)], sem).wait()

        def blendout(g, cv, lov, hiv, ov):
            base = wid * per_w + g * CHUNK

            def blend_body(t, _):
                w16 = cv[pl.ds(t * L, L)]
                for k in range(L):
                    i = t * L + k
                    wv = _bcast_lane(w16, k)
                    for d in range(dim // L):
                        lo = lov[i, pl.ds(d * L, L)]
                        hi = hiv[i, pl.ds(d * L, L)]
                        ov[i, pl.ds(d * L, L)] = lo + wv * (hi - lo)
                return 0

            lax.fori_loop(0, CHUNK // L, blend_body, 0)
            pltpu.sync_copy(ov, out_hbm.at[pl.ds(base, CHUNK)])

        set0 = (c0, il0, ih0, lo0, hi0, s0)
        set1 = (c1, il1, ih1, lo1, hi1, s1)

        prep(0, *set0)

        def step(s, _):
            prep(2 * s + 1, *set1)
            waitg(*set0[1:])
            blendout(2 * s, c0, lo0, hi0, o0)
            prep(2 * s + 2, *set0)
            waitg(*set1[1:])
            blendout(2 * s + 1, c1, lo1, hi1, o1)
            return 0

        lax.fori_loop(0, n_chunks // 2, step, 0)

    return impl


def kernel(continuous_idx, W):
    batch, fields = continuous_idx.shape
    vocab, dim = W.shape
    n_total = batch * fields
    impl = _build(n_total, vocab, dim)
    out = impl(continuous_idx.reshape(n_total), W)
    return out.reshape(batch, fields, dim)


# speedup vs baseline: 1.4622x; 1.0032x over previous
import functools

import jax
import jax.numpy as jnp
from jax import lax
from jax.experimental import pallas as pl
from jax.experimental.pallas import tpu as pltpu
from jax.experimental.pallas import tpu_sc as plsc

L = 16
NC, NS = 2, 16
NW = NC * NS
IDXROW = 128


def _bcast_lane(v, k):
    return lax.gather(
        v, jnp.full((L, 1), k, jnp.int32),
        lax.GatherDimensionNumbers(
            offset_dims=(), collapsed_slice_dims=(0,), start_index_map=(0,)),
        slice_sizes=(1,),
        mode=lax.GatherScatterMode.PROMISE_IN_BOUNDS)


@functools.lru_cache(maxsize=None)
def _build(batch, fields, vocab, dim):
    bw = batch // NW
    kstream = bw // IDXROW
    n_chunks = fields
    max_idx = vocab - 1

    mesh = plsc.VectorSubcoreMesh(core_axis_name="c", subcore_axis_name="s")

    @functools.partial(
        pl.kernel,
        out_type=jax.ShapeDtypeStruct((batch, fields, dim), jnp.float32),
        mesh=mesh,
        compiler_params=pltpu.CompilerParams(use_tc_tiling_on_sc=False),
        scratch_types=[
            pltpu.VMEM((bw,), jnp.float32),
            pltpu.VMEM((bw,), jnp.float32),
            pltpu.VMEM((kstream, IDXROW), jnp.int32),
            pltpu.VMEM((kstream, IDXROW), jnp.int32),
            pltpu.VMEM((kstream, IDXROW), jnp.int32),
            pltpu.VMEM((kstream, IDXROW), jnp.int32),
            pltpu.VMEM((bw, dim), jnp.float32),
            pltpu.VMEM((bw, dim), jnp.float32),
            pltpu.VMEM((bw, dim), jnp.float32),
            pltpu.VMEM((bw, dim), jnp.float32),
            pltpu.VMEM((bw, 1, dim), jnp.float32),
            pltpu.VMEM((bw, 1, dim), jnp.float32),
            pltpu.SemaphoreType.DMA,
            pltpu.SemaphoreType.DMA,
        ],
    )
    def impl(cont_hbm, w_hbm, out_hbm, c0, c1, il0, ih0, il1, ih1,
             lo0, hi0, lo1, hi1, o0, o1, s0, s1):
        wid = lax.axis_index("s") * NC + lax.axis_index("c")
        b0 = wid * bw

        def prep(f, cv, ilv, ihv, lov, hiv, sem):
            @pl.when(f < n_chunks)
            def _():
                pltpu.sync_copy(cont_hbm.at[f, pl.ds(b0, bw)], cv)

                def idx_body(t, _):
                    x = cv[pl.ds(t * L, L)]
                    il = x.astype(jnp.int32)
                    w = x - il.astype(jnp.float32)
                    ih = jnp.minimum(il + 1, max_idx)
                    r = t // (IDXROW // L)
                    c = (t % (IDXROW // L)) * L
                    ilv[r, pl.ds(c, L)] = il
                    ihv[r, pl.ds(c, L)] = ih
                    cv[pl.ds(t * L, L)] = w
                    return 0

                lax.fori_loop(0, bw // L, idx_body, 0)
                for j in range(kstream):
                    pltpu.async_copy(
                        w_hbm.at[ilv.at[j]],
                        lov.at[pl.ds(j * IDXROW, IDXROW)], sem)
                    pltpu.async_copy(
                        w_hbm.at[ihv.at[j]],
                        hiv.at[pl.ds(j * IDXROW, IDXROW)], sem)

        def waitg(ilv, ihv, lov, hiv, sem):
            for j in range(kstream):
                pltpu.make_async_copy(
                    w_hbm.at[ilv.at[j]],
                    lov.at[pl.ds(j * IDXROW, IDXROW)], sem).wait()
                pltpu.make_async_copy(
                    w_hbm.at[ihv.at[j]],
                    hiv.at[pl.ds(j * IDXROW, IDXROW)], sem).wait()

        def blendout(f, cv, lov, hiv, ov):
            def blend_body(t, _):
                w16 = cv[pl.ds(t * L, L)]
                for k in range(L):
                    i = t * L + k
                    wv = _bcast_lane(w16, k)
                    for d in range(dim // L):
                        lo = lov[i, pl.ds(d * L, L)]
                        hi = hiv[i, pl.ds(d * L, L)]
                        ov[i, 0, pl.ds(d * L, L)] = lo + wv * (hi - lo)
                return 0

            lax.fori_loop(0, bw // L, blend_body, 0)
            pltpu.sync_copy(ov, out_hbm.at[pl.ds(b0, bw), pl.ds(f, 1)])

        set0 = (c0, il0, ih0, lo0, hi0, s0)
        set1 = (c1, il1, ih1, lo1, hi1, s1)

        prep(0, *set0)

        def step(s, _):
            prep(2 * s + 1, *set1)
            waitg(*set0[1:])
            blendout(2 * s, c0, lo0, hi0, o0)
            prep(2 * s + 2, *set0)
            waitg(*set1[1:])
            blendout(2 * s + 1, c1, lo1, hi1, o1)
            return 0

        lax.fori_loop(0, n_chunks // 2, step, 0)

    return impl


def kernel(continuous_idx, W):
    batch, fields = continuous_idx.shape
    vocab, dim = W.shape
    impl = _build(batch, fields, vocab, dim)
    return impl(continuous_idx.T, W)


# speedup vs baseline: 1.4869x; 1.0169x over previous
import functools

import jax
import jax.numpy as jnp
from jax import lax
from jax.experimental import pallas as pl
from jax.experimental.pallas import tpu as pltpu
from jax.experimental.pallas import tpu_sc as plsc

L = 16
NC, NS = 2, 16
NW = NC * NS
IDXROW = 128


def _bcast_lane(v, k):
    return lax.gather(
        v, jnp.full((L, 1), k, jnp.int32),
        lax.GatherDimensionNumbers(
            offset_dims=(), collapsed_slice_dims=(0,), start_index_map=(0,)),
        slice_sizes=(1,),
        mode=lax.GatherScatterMode.PROMISE_IN_BOUNDS)


@functools.lru_cache(maxsize=None)
def _build(batch, fields, vocab, dim):
    bw = batch // NW
    kstream = bw // IDXROW
    n_chunks = fields
    max_idx = vocab - 1

    mesh = plsc.VectorSubcoreMesh(core_axis_name="c", subcore_axis_name="s")

    @functools.partial(
        pl.kernel,
        out_type=jax.ShapeDtypeStruct((batch, fields, dim), jnp.float32),
        mesh=mesh,
        compiler_params=pltpu.CompilerParams(use_tc_tiling_on_sc=False),
        scratch_types=[
            pltpu.VMEM((bw,), jnp.float32),
            pltpu.VMEM((bw,), jnp.float32),
            pltpu.VMEM((kstream, IDXROW), jnp.int32),
            pltpu.VMEM((kstream, IDXROW), jnp.int32),
            pltpu.VMEM((kstream, IDXROW), jnp.int32),
            pltpu.VMEM((kstream, IDXROW), jnp.int32),
            pltpu.VMEM((bw, dim), jnp.float32),
            pltpu.VMEM((bw, dim), jnp.float32),
            pltpu.VMEM((bw, dim), jnp.float32),
            pltpu.VMEM((bw, dim), jnp.float32),
            pltpu.VMEM((bw, 1, dim), jnp.float32),
            pltpu.VMEM((bw, 1, dim), jnp.float32),
            pltpu.SemaphoreType.DMA,
            pltpu.SemaphoreType.DMA,
        ],
    )
    def impl(cont_hbm, w_hbm, out_hbm, c0, c1, il0, ih0, il1, ih1,
             lo0, hi0, lo1, hi1, o0, o1, s0, s1):
        wid = lax.axis_index("s") * NC + lax.axis_index("c")
        b0 = wid * bw

        def prep(f, cv, ilv, ihv, lov, hiv, sem):
            @pl.when(f < n_chunks)
            def _():
                pltpu.sync_copy(cont_hbm.at[f, pl.ds(b0, bw)], cv)

                def idx_body(t, _):
                    x = cv[pl.ds(t * L, L)]
                    il = x.astype(jnp.int32)
                    w = x - il.astype(jnp.float32)
                    ih = jnp.minimum(il + 1, max_idx)
                    r = t // (IDXROW // L)
                    c = (t % (IDXROW // L)) * L
                    ilv[r, pl.ds(c, L)] = il << 2
                    ihv[r, pl.ds(c, L)] = ih << 2
                    cv[pl.ds(t * L, L)] = w
                    return 0

                lax.fori_loop(0, bw // L, idx_body, 0)
                for j in range(kstream):
                    pltpu.async_copy(
                        w_hbm.at[ilv.at[j]],
                        lov.at[pl.ds(j * IDXROW, IDXROW)], sem)
                    pltpu.async_copy(
                        w_hbm.at[ihv.at[j]],
                        hiv.at[pl.ds(j * IDXROW, IDXROW)], sem)

        def waitg(ilv, ihv, lov, hiv, sem):
            for j in range(kstream):
                pltpu.make_async_copy(
                    w_hbm.at[ilv.at[j]],
                    lov.at[pl.ds(j * IDXROW, IDXROW)], sem).wait()
                pltpu.make_async_copy(
                    w_hbm.at[ihv.at[j]],
                    hiv.at[pl.ds(j * IDXROW, IDXROW)], sem).wait()

        def blendout(f, cv, lov, hiv, ov):
            def blend_body(t, _):
                w16 = cv[pl.ds(t * L, L)]
                for k in range(L):
                    i = t * L + k
                    wv = _bcast_lane(w16, k)
                    for d in range(dim // L):
                        lo = lov[i, pl.ds(d * L, L)]
                        hi = hiv[i, pl.ds(d * L, L)]
                        ov[i, 0, pl.ds(d * L, L)] = lo + wv * (hi - lo)
                return 0

            lax.fori_loop(0, bw // L, blend_body, 0)
            pltpu.sync_copy(ov, out_hbm.at[pl.ds(b0, bw), pl.ds(f, 1)])

        set0 = (c0, il0, ih0, lo0, hi0, s0)
        set1 = (c1, il1, ih1, lo1, hi1, s1)

        prep(0, *set0)

        def step(s, _):
            prep(2 * s + 1, *set1)
            waitg(*set0[1:])
            blendout(2 * s, c0, lo0, hi0, o0)
            prep(2 * s + 2, *set0)
            waitg(*set1[1:])
            blendout(2 * s + 1, c1, lo1, hi1, o1)
            return 0

        lax.fori_loop(0, n_chunks // 2, step, 0)

    return impl


def kernel(continuous_idx, W):
    batch, fields = continuous_idx.shape
    vocab, dim = W.shape
    impl = _build(batch, fields, vocab, dim)
    w_pad = jnp.pad(W, ((0, 0), (0, 3 * dim)))
    return impl(continuous_idx.T, w_pad.reshape(4 * vocab, dim))
